# baseline (speedup 1.0000x reference)
import functools

import jax
import jax.numpy as jnp
from jax import lax
from jax.experimental import pallas as pl
from jax.experimental.pallas import tpu as pltpu
from jax.experimental.pallas import tpu_sc as plsc

B, C, H, W = 4, 96, 224, 224
HW = H * W
CP1 = C + 1
NIMG = B * CP1
PAD = 240
ACC_LEN = HW + 2 * PAD
NC, NS = 2, 16
NW = NC * NS
CH = 1792
NCHUNK = HW // CH
NUNIT_B = 49
NB = 2
NUNIT = NB * NUNIT_B
NT = (NUNIT + NW - 1) // NW


def _pre_body(f0_ref, f1_ref, m_ref, idx_ref, w0_ref, w1_ref, w2_ref, w3_ref):
    shp = (1, 1, H, W)
    xs = lax.broadcasted_iota(jnp.int32, shp, 3).astype(jnp.float32)
    ys = lax.broadcasted_iota(jnp.int32, shp, 2).astype(jnp.float32)
    fx = xs + f0_ref[...]
    fy = ys + f1_ref[...]
    nwx = jnp.floor(fx)
    nwy = jnp.floor(fy)
    ix = nwx.astype(jnp.int32)
    iy = nwy.astype(jnp.int32)
    tx = fx - nwx
    ty = fy - nwy
    ex = jnp.exp(m_ref[...])
    vx0 = (ix >= 0) & (ix < W)
    vx1 = (ix >= -1) & (ix < W - 1)
    vy0 = (iy >= 0) & (iy < H)
    vy1 = (iy >= -1) & (iy < H - 1)
    zero = jnp.zeros(shp, jnp.float32)
    w0_ref[...] = jnp.where(vx0 & vy0, (1.0 - tx) * (1.0 - ty) * ex, zero)
    w1_ref[...] = jnp.where(vx1 & vy0, tx * (1.0 - ty) * ex, zero)
    w2_ref[...] = jnp.where(vx0 & vy1, (1.0 - tx) * ty * ex, zero)
    w3_ref[...] = jnp.where(vx1 & vy1, tx * ty * ex, zero)
    oob = (ix < -1) | (ix >= W) | (iy < -1) | (iy >= H)
    idx_ref[...] = jnp.where(oob, 0, iy * W + ix + PAD)


def _precompute(tenFlow, tenMetric):
    img_spec = lambda ch: pl.BlockSpec((1, 1, H, W), lambda b, ch=ch: (b, ch, 0, 0))
    out4 = jax.ShapeDtypeStruct((B, 1, H, W), jnp.float32)
    return pl.pallas_call(
        _pre_body,
        grid=(B,),
        in_specs=[img_spec(0), img_spec(1), img_spec(0)],
        out_specs=[img_spec(0)] * 5,
        out_shape=[jax.ShapeDtypeStruct((B, 1, H, W), jnp.int32),
                   out4, out4, out4, out4],
    )(tenFlow, tenFlow, tenMetric)


def _sc_body(inp_hbm, idx_hbm, w0_hbm, w1_hbm, w2_hbm, w3_hbm, out_hbm,
             acc0, acc1,
             val0b0, val1b0, idxb0, w0b0, w1b0, w2b0, w3b0,
             val0b1, val1b1, idxb1, w0b1, w1b1, w2b1, w3b1,
             sem0, sem1):
    wid = lax.axis_index("s") * NC + lax.axis_index("c")
    slots = ((val0b0, val1b0, idxb0, w0b0, w1b0, w2b0, w3b0, sem0),
             (val0b1, val1b1, idxb1, w0b1, w1b1, w2b1, w3b1, sem1))

    def start_chunk(k, slot, b, c0, single):
        val0b, val1b, idxb, w0b, w1b, w2b, w3b, sem = slots[slot]
        s = k * CH

        @pl.when(jnp.logical_not(single))
        def _():
            pltpu.async_copy(inp_hbm.at[b * C + c0, pl.ds(s, CH)], val0b, sem)
            pltpu.async_copy(inp_hbm.at[b * C + c0 + 1, pl.ds(s, CH)], val1b, sem)
        pltpu.async_copy(idx_hbm.at[b, pl.ds(s, CH)], idxb, sem)
        pltpu.async_copy(w0_hbm.at[b, pl.ds(s, CH)], w0b, sem)
        pltpu.async_copy(w1_hbm.at[b, pl.ds(s, CH)], w1b, sem)
        pltpu.async_copy(w2_hbm.at[b, pl.ds(s, CH)], w2b, sem)
        pltpu.async_copy(w3_hbm.at[b, pl.ds(s, CH)], w3b, sem)

    def wait_chunk(k, slot, b, c0, single):
        val0b, val1b, idxb, w0b, w1b, w2b, w3b, sem = slots[slot]
        s = k * CH

        @pl.when(jnp.logical_not(single))
        def _():
            pltpu.make_async_copy(
                inp_hbm.at[b * C + c0, pl.ds(s, CH)], val0b, sem).wait()
            pltpu.make_async_copy(
                inp_hbm.at[b * C + c0 + 1, pl.ds(s, CH)], val1b, sem).wait()
        pltpu.make_async_copy(idx_hbm.at[b, pl.ds(s, CH)], idxb, sem).wait()
        pltpu.make_async_copy(w0_hbm.at[b, pl.ds(s, CH)], w0b, sem).wait()
        pltpu.make_async_copy(w1_hbm.at[b, pl.ds(s, CH)], w1b, sem).wait()
        pltpu.make_async_copy(w2_hbm.at[b, pl.ds(s, CH)], w2b, sem).wait()
        pltpu.make_async_copy(w3_hbm.at[b, pl.ds(s, CH)], w3b, sem).wait()

    def compute_chunk(slot):
        val0b, val1b, idxb, w0b, w1b, w2b, w3b, sem = slots[slot]

        @plsc.parallel_loop(0, CH // 16, 1, unroll=8)
        def _g(g):
            o = g * 16
            v0 = val0b[pl.ds(o, 16)]
            v1 = val1b[pl.ds(o, 16)]
            ii = idxb[pl.ds(o, 16)]
            i1 = ii + 1
            i2 = ii + W
            i3 = ii + (W + 1)
            w0 = w0b[pl.ds(o, 16)]
            w1 = w1b[pl.ds(o, 16)]
            w2 = w2b[pl.ds(o, 16)]
            w3 = w3b[pl.ds(o, 16)]
            plsc.addupdate_scatter(acc0, [ii], v0 * w0)
            plsc.addupdate_scatter(acc1, [ii], v1 * w0)
            plsc.addupdate_scatter(acc0, [i1], v0 * w1)
            plsc.addupdate_scatter(acc1, [i1], v1 * w1)
            plsc.addupdate_scatter(acc0, [i2], v0 * w2)
            plsc.addupdate_scatter(acc1, [i2], v1 * w2)
            plsc.addupdate_scatter(acc0, [i3], v0 * w3)
            plsc.addupdate_scatter(acc1, [i3], v1 * w3)

    def unit_loop(t, carry):
        u = wid + t * NW

        @pl.when(u < NUNIT)
        def _():
            b = (u >= NUNIT_B).astype(jnp.int32)
            j = u - b * NUNIT_B
            c0 = 2 * j
            single = j == (NUNIT_B - 1)

            start_chunk(0, 0, b, c0, single)
            start_chunk(1, 1, b, c0, single)

            @pl.when(single)
            def _():
                for slot in range(2):
                    for vb in (slots[slot][0], slots[slot][1]):

                        @plsc.parallel_loop(0, CH // 16, 1, unroll=8)
                        def _o(i):
                            vb[pl.ds(i * 16, 16)] = jnp.ones((16,), jnp.float32)

            @plsc.parallel_loop(0, ACC_LEN // 16, 1, unroll=8)
            def _z(i):
                acc0[pl.ds(i * 16, 16)] = jnp.zeros((16,), jnp.float32)
                acc1[pl.ds(i * 16, 16)] = jnp.zeros((16,), jnp.float32)

            def chunk_loop(kk, carry2):
                for slot in range(2):
                    k = kk * 2 + slot
                    wait_chunk(k, slot, b, c0, single)
                    compute_chunk(slot)

                    @pl.when(k + 2 < NCHUNK)
                    def _():
                        start_chunk(k + 2, slot, b, c0, single)
                return carry2
            lax.fori_loop(0, NCHUNK // 2, chunk_loop, 0)

            img0 = b * CP1 + c0
            cp0 = pltpu.async_copy(acc0.at[pl.ds(PAD, HW)], out_hbm.at[img0],
                                   sem0)

            @pl.when(jnp.logical_not(single))
            def _():
                pltpu.async_copy(acc1.at[pl.ds(PAD, HW)], out_hbm.at[img0 + 1],
                                 sem1)
            cp0.wait()

            @pl.when(jnp.logical_not(single))
            def _():
                pltpu.make_async_copy(acc1.at[pl.ds(PAD, HW)],
                                      out_hbm.at[img0 + 1], sem1).wait()
        return carry
    lax.fori_loop(0, NT, unit_loop, 0)


@functools.lru_cache(maxsize=1)
def _sc_scatter():
  return functools.partial(
    pl.kernel,
    out_type=jax.ShapeDtypeStruct((NB * CP1, HW), jnp.float32),
    mesh=plsc.VectorSubcoreMesh(core_axis_name="c", subcore_axis_name="s",
                                num_cores=NC, num_subcores=NS),
    compiler_params=pltpu.CompilerParams(needs_layout_passes=False,
                                         use_tc_tiling_on_sc=False),
    scratch_types=(
        [pltpu.VMEM((ACC_LEN,), jnp.float32), pltpu.VMEM((ACC_LEN,), jnp.float32)]
        + [pltpu.VMEM((CH,), jnp.float32),
           pltpu.VMEM((CH,), jnp.float32),
           pltpu.VMEM((CH,), jnp.int32),
           pltpu.VMEM((CH,), jnp.float32),
           pltpu.VMEM((CH,), jnp.float32),
           pltpu.VMEM((CH,), jnp.float32),
           pltpu.VMEM((CH,), jnp.float32)] * 2
        + [pltpu.SemaphoreType.DMA, pltpu.SemaphoreType.DMA]
    ),
  )(_sc_body)


def _norm_body(num_ref, den_ref, out_ref):
    den = den_ref[...]
    den = jnp.where(den == 0.0, 1.0, den)
    out_ref[...] = num_ref[...] / den


CG = 16


def _normalize(acc4):
    return pl.pallas_call(
        _norm_body,
        grid=(NB, C // CG),
        in_specs=[pl.BlockSpec((1, CG, H, W), lambda b, c: (b, c, 0, 0)),
                  pl.BlockSpec((1, 1, H, W), lambda b, c: (b, C, 0, 0))],
        out_specs=pl.BlockSpec((1, CG, H, W), lambda b, c: (b, c, 0, 0)),
        out_shape=jax.ShapeDtypeStruct((NB, C, H, W), jnp.float32),
    )(acc4, acc4)


def kernel(tenInput, tenFlow, tenMetric):
    idx4, w04, w14, w24, w34 = _precompute(tenFlow, tenMetric)
    idx = idx4.reshape(B, HW)
    ws = [w.reshape(B, HW) for w in (w04, w14, w24, w34)]
    halves = []
    for h in range(B // NB):
        b0 = h * NB
        acc = _sc_scatter()(
            tenInput[b0:b0 + NB].reshape(NB * C, HW),
            idx[b0:b0 + NB],
            *[w[b0:b0 + NB] for w in ws],
        )
        halves.append(_normalize(acc.reshape(NB, CP1, H, W)))
    return jnp.concatenate(halves, axis=0)


# speedup vs baseline: 1.0191x; 1.0191x over previous
import functools

import jax
import jax.numpy as jnp
from jax import lax
from jax.experimental import pallas as pl
from jax.experimental.pallas import tpu as pltpu
from jax.experimental.pallas import tpu_sc as plsc

B, C, H, W = 4, 96, 224, 224
HW = H * W
CP1 = C + 1
NIMG = B * CP1
PAD = 240
ACC_LEN = HW + 2 * PAD
NC, NS = 2, 16
NW = NC * NS
CH = 1792
NCHUNK = HW // CH
NUNIT_B = 49
NB = 2
NUNIT = NB * NUNIT_B
NT = (NUNIT + NW - 1) // NW


def _pre_body(f0_ref, f1_ref, m_ref, idx_ref, w0_ref, w1_ref, w2_ref, w3_ref):
    shp = (1, 1, H, W)
    xs = lax.broadcasted_iota(jnp.int32, shp, 3).astype(jnp.float32)
    ys = lax.broadcasted_iota(jnp.int32, shp, 2).astype(jnp.float32)
    fx = xs + f0_ref[...]
    fy = ys + f1_ref[...]
    nwx = jnp.floor(fx)
    nwy = jnp.floor(fy)
    ix = nwx.astype(jnp.int32)
    iy = nwy.astype(jnp.int32)
    tx = fx - nwx
    ty = fy - nwy
    ex = jnp.exp(m_ref[...])
    vx0 = (ix >= 0) & (ix < W)
    vx1 = (ix >= -1) & (ix < W - 1)
    vy0 = (iy >= 0) & (iy < H)
    vy1 = (iy >= -1) & (iy < H - 1)
    zero = jnp.zeros(shp, jnp.float32)
    w0_ref[...] = jnp.where(vx0 & vy0, (1.0 - tx) * (1.0 - ty) * ex, zero)
    w1_ref[...] = jnp.where(vx1 & vy0, tx * (1.0 - ty) * ex, zero)
    w2_ref[...] = jnp.where(vx0 & vy1, (1.0 - tx) * ty * ex, zero)
    w3_ref[...] = jnp.where(vx1 & vy1, tx * ty * ex, zero)
    oob = (ix < -1) | (ix >= W) | (iy < -1) | (iy >= H)
    idx_ref[...] = jnp.where(oob, 0, iy * W + ix + PAD)


def _precompute(tenFlow, tenMetric):
    img_spec = lambda ch: pl.BlockSpec((1, 1, H, W), lambda b, ch=ch: (b, ch, 0, 0))
    out4 = jax.ShapeDtypeStruct((B, 1, H, W), jnp.float32)
    return pl.pallas_call(
        _pre_body,
        grid=(B,),
        in_specs=[img_spec(0), img_spec(1), img_spec(0)],
        out_specs=[img_spec(0)] * 5,
        out_shape=[jax.ShapeDtypeStruct((B, 1, H, W), jnp.int32),
                   out4, out4, out4, out4],
    )(tenFlow, tenFlow, tenMetric)


def _sc_body(boff, inp_hbm, idx_hbm, w0_hbm, w1_hbm, w2_hbm, w3_hbm, out_hbm,
             acc0, acc1,
             val0b0, val1b0, idxb0, w0b0, w1b0, w2b0, w3b0,
             val0b1, val1b1, idxb1, w0b1, w1b1, w2b1, w3b1,
             sem0, sem1):
    wid = lax.axis_index("s") * NC + lax.axis_index("c")
    slots = ((val0b0, val1b0, idxb0, w0b0, w1b0, w2b0, w3b0, sem0),
             (val0b1, val1b1, idxb1, w0b1, w1b1, w2b1, w3b1, sem1))

    def start_chunk(k, slot, b, c0, single):
        val0b, val1b, idxb, w0b, w1b, w2b, w3b, sem = slots[slot]
        s = k * CH

        @pl.when(jnp.logical_not(single))
        def _():
            pltpu.async_copy(inp_hbm.at[b * C + c0, pl.ds(s, CH)], val0b, sem)
            pltpu.async_copy(inp_hbm.at[b * C + c0 + 1, pl.ds(s, CH)], val1b, sem)
        pltpu.async_copy(idx_hbm.at[b, pl.ds(s, CH)], idxb, sem)
        pltpu.async_copy(w0_hbm.at[b, pl.ds(s, CH)], w0b, sem)
        pltpu.async_copy(w1_hbm.at[b, pl.ds(s, CH)], w1b, sem)
        pltpu.async_copy(w2_hbm.at[b, pl.ds(s, CH)], w2b, sem)
        pltpu.async_copy(w3_hbm.at[b, pl.ds(s, CH)], w3b, sem)

    def wait_chunk(k, slot, b, c0, single):
        val0b, val1b, idxb, w0b, w1b, w2b, w3b, sem = slots[slot]
        s = k * CH

        @pl.when(jnp.logical_not(single))
        def _():
            pltpu.make_async_copy(
                inp_hbm.at[b * C + c0, pl.ds(s, CH)], val0b, sem).wait()
            pltpu.make_async_copy(
                inp_hbm.at[b * C + c0 + 1, pl.ds(s, CH)], val1b, sem).wait()
        pltpu.make_async_copy(idx_hbm.at[b, pl.ds(s, CH)], idxb, sem).wait()
        pltpu.make_async_copy(w0_hbm.at[b, pl.ds(s, CH)], w0b, sem).wait()
        pltpu.make_async_copy(w1_hbm.at[b, pl.ds(s, CH)], w1b, sem).wait()
        pltpu.make_async_copy(w2_hbm.at[b, pl.ds(s, CH)], w2b, sem).wait()
        pltpu.make_async_copy(w3_hbm.at[b, pl.ds(s, CH)], w3b, sem).wait()

    def compute_chunk(slot):
        val0b, val1b, idxb, w0b, w1b, w2b, w3b, sem = slots[slot]

        @plsc.parallel_loop(0, CH // 16, 1, unroll=8)
        def _g(g):
            o = g * 16
            v0 = val0b[pl.ds(o, 16)]
            v1 = val1b[pl.ds(o, 16)]
            ii = idxb[pl.ds(o, 16)]
            i1 = ii + 1
            i2 = ii + W
            i3 = ii + (W + 1)
            w0 = w0b[pl.ds(o, 16)]
            w1 = w1b[pl.ds(o, 16)]
            w2 = w2b[pl.ds(o, 16)]
            w3 = w3b[pl.ds(o, 16)]
            plsc.addupdate_scatter(acc0, [ii], v0 * w0)
            plsc.addupdate_scatter(acc1, [ii], v1 * w0)
            plsc.addupdate_scatter(acc0, [i1], v0 * w1)
            plsc.addupdate_scatter(acc1, [i1], v1 * w1)
            plsc.addupdate_scatter(acc0, [i2], v0 * w2)
            plsc.addupdate_scatter(acc1, [i2], v1 * w2)
            plsc.addupdate_scatter(acc0, [i3], v0 * w3)
            plsc.addupdate_scatter(acc1, [i3], v1 * w3)

    def unit_loop(t, carry):
        u = wid + t * NW

        @pl.when(u < NUNIT)
        def _():
            bl = (u >= NUNIT_B).astype(jnp.int32)
            b = bl + boff
            j = u - bl * NUNIT_B
            c0 = 2 * j
            single = j == (NUNIT_B - 1)

            start_chunk(0, 0, b, c0, single)
            start_chunk(1, 1, b, c0, single)

            @pl.when(single)
            def _():
                for slot in range(2):
                    for vb in (slots[slot][0], slots[slot][1]):

                        @plsc.parallel_loop(0, CH // 16, 1, unroll=8)
                        def _o(i):
                            vb[pl.ds(i * 16, 16)] = jnp.ones((16,), jnp.float32)

            @plsc.parallel_loop(0, ACC_LEN // 16, 1, unroll=8)
            def _z(i):
                acc0[pl.ds(i * 16, 16)] = jnp.zeros((16,), jnp.float32)
                acc1[pl.ds(i * 16, 16)] = jnp.zeros((16,), jnp.float32)

            def chunk_loop(kk, carry2):
                for slot in range(2):
                    k = kk * 2 + slot
                    wait_chunk(k, slot, b, c0, single)
                    compute_chunk(slot)

                    @pl.when(k + 2 < NCHUNK)
                    def _():
                        start_chunk(k + 2, slot, b, c0, single)
                return carry2
            lax.fori_loop(0, NCHUNK // 2, chunk_loop, 0)

            img0 = bl * CP1 + c0
            cp0 = pltpu.async_copy(acc0.at[pl.ds(PAD, HW)], out_hbm.at[img0],
                                   sem0)

            @pl.when(jnp.logical_not(single))
            def _():
                pltpu.async_copy(acc1.at[pl.ds(PAD, HW)], out_hbm.at[img0 + 1],
                                 sem1)
            cp0.wait()

            @pl.when(jnp.logical_not(single))
            def _():
                pltpu.make_async_copy(acc1.at[pl.ds(PAD, HW)],
                                      out_hbm.at[img0 + 1], sem1).wait()
        return carry
    lax.fori_loop(0, NT, unit_loop, 0)


@functools.lru_cache(maxsize=2)
def _sc_scatter(boff):
  return functools.partial(
    pl.kernel,
    out_type=jax.ShapeDtypeStruct((NB * CP1, HW), jnp.float32),
    mesh=plsc.VectorSubcoreMesh(core_axis_name="c", subcore_axis_name="s",
                                num_cores=NC, num_subcores=NS),
    compiler_params=pltpu.CompilerParams(needs_layout_passes=False,
                                         use_tc_tiling_on_sc=False),
    scratch_types=(
        [pltpu.VMEM((ACC_LEN,), jnp.float32), pltpu.VMEM((ACC_LEN,), jnp.float32)]
        + [pltpu.VMEM((CH,), jnp.float32),
           pltpu.VMEM((CH,), jnp.float32),
           pltpu.VMEM((CH,), jnp.int32),
           pltpu.VMEM((CH,), jnp.float32),
           pltpu.VMEM((CH,), jnp.float32),
           pltpu.VMEM((CH,), jnp.float32),
           pltpu.VMEM((CH,), jnp.float32)] * 2
        + [pltpu.SemaphoreType.DMA, pltpu.SemaphoreType.DMA]
    ),
  )(functools.partial(_sc_body, boff))


def _norm_body(num_ref, den_ref, out_ref):
    den = den_ref[...]
    den = jnp.where(den == 0.0, 1.0, den)
    out_ref[...] = num_ref[...] / den


CG = 16


def _normalize(acc4):
    return pl.pallas_call(
        _norm_body,
        grid=(NB, C // CG),
        in_specs=[pl.BlockSpec((1, CG, H, W), lambda b, c: (b, c, 0, 0)),
                  pl.BlockSpec((1, 1, H, W), lambda b, c: (b, C, 0, 0))],
        out_specs=pl.BlockSpec((1, CG, H, W), lambda b, c: (b, c, 0, 0)),
        out_shape=jax.ShapeDtypeStruct((NB, C, H, W), jnp.float32),
    )(acc4, acc4)


def kernel(tenInput, tenFlow, tenMetric):
    idx4, w04, w14, w24, w34 = _precompute(tenFlow, tenMetric)
    idx = idx4.reshape(B, HW)
    ws = [w.reshape(B, HW) for w in (w04, w14, w24, w34)]
    inp2d = tenInput.reshape(B * C, HW)
    halves = []
    for h in range(B // NB):
        acc = _sc_scatter(h * NB)(inp2d, idx, *ws)
        halves.append(_normalize(acc.reshape(NB, CP1, H, W)))
    return jnp.concatenate(halves, axis=0)


# speedup vs baseline: 1.1204x; 1.0994x over previous
import functools

import jax
import jax.numpy as jnp
from jax import lax
from jax.experimental import pallas as pl
from jax.experimental.pallas import tpu as pltpu
from jax.experimental.pallas import tpu_sc as plsc

B, C, H, W = 4, 96, 224, 224
HW = H * W
CP1 = C + 1
NIMG = B * CP1
PAD = 240
ACC_LEN = HW + 2 * PAD
NC, NS = 2, 16
NW = NC * NS
CH = 1792
NCHUNK = HW // CH
NUNIT_B = 49
NB = 4
NUNIT = NB * NUNIT_B
NT = (NUNIT + NW - 1) // NW


def _pre_body(f0_ref, f1_ref, m_ref, idx_ref, w0_ref, w1_ref, w2_ref, w3_ref):
    shp = (1, 1, H, W)
    xs = lax.broadcasted_iota(jnp.int32, shp, 3).astype(jnp.float32)
    ys = lax.broadcasted_iota(jnp.int32, shp, 2).astype(jnp.float32)
    fx = xs + f0_ref[...]
    fy = ys + f1_ref[...]
    nwx = jnp.floor(fx)
    nwy = jnp.floor(fy)
    ix = nwx.astype(jnp.int32)
    iy = nwy.astype(jnp.int32)
    tx = fx - nwx
    ty = fy - nwy
    ex = jnp.exp(m_ref[...])
    vx0 = (ix >= 0) & (ix < W)
    vx1 = (ix >= -1) & (ix < W - 1)
    vy0 = (iy >= 0) & (iy < H)
    vy1 = (iy >= -1) & (iy < H - 1)
    zero = jnp.zeros(shp, jnp.float32)
    w0_ref[...] = jnp.where(vx0 & vy0, (1.0 - tx) * (1.0 - ty) * ex, zero)
    w1_ref[...] = jnp.where(vx1 & vy0, tx * (1.0 - ty) * ex, zero)
    w2_ref[...] = jnp.where(vx0 & vy1, (1.0 - tx) * ty * ex, zero)
    w3_ref[...] = jnp.where(vx1 & vy1, tx * ty * ex, zero)
    oob = (ix < -1) | (ix >= W) | (iy < -1) | (iy >= H)
    idx_ref[...] = jnp.where(oob, 0, iy * W + ix + PAD)


def _precompute(tenFlow, tenMetric):
    img_spec = lambda ch: pl.BlockSpec((1, 1, H, W), lambda b, ch=ch: (b, ch, 0, 0))
    out4 = jax.ShapeDtypeStruct((B, 1, H, W), jnp.float32)
    return pl.pallas_call(
        _pre_body,
        grid=(B,),
        in_specs=[img_spec(0), img_spec(1), img_spec(0)],
        out_specs=[img_spec(0)] * 5,
        out_shape=[jax.ShapeDtypeStruct((B, 1, H, W), jnp.int32),
                   out4, out4, out4, out4],
    )(tenFlow, tenFlow, tenMetric)


def _sc_body(boff, inp_hbm, idx_hbm, w0_hbm, w1_hbm, w2_hbm, w3_hbm, out_hbm,
             acc0, acc1,
             val0b0, val1b0, idxb0, w0b0, w1b0, w2b0, w3b0,
             val0b1, val1b1, idxb1, w0b1, w1b1, w2b1, w3b1,
             sem0, sem1):
    wid = lax.axis_index("s") * NC + lax.axis_index("c")
    slots = ((val0b0, val1b0, idxb0, w0b0, w1b0, w2b0, w3b0, sem0),
             (val0b1, val1b1, idxb1, w0b1, w1b1, w2b1, w3b1, sem1))

    def start_chunk(k, slot, b, c0, single):
        val0b, val1b, idxb, w0b, w1b, w2b, w3b, sem = slots[slot]
        s = k * CH

        @pl.when(jnp.logical_not(single))
        def _():
            pltpu.async_copy(inp_hbm.at[b * C + c0, pl.ds(s, CH)], val0b, sem)
            pltpu.async_copy(inp_hbm.at[b * C + c0 + 1, pl.ds(s, CH)], val1b, sem)
        pltpu.async_copy(idx_hbm.at[b, pl.ds(s, CH)], idxb, sem)
        pltpu.async_copy(w0_hbm.at[b, pl.ds(s, CH)], w0b, sem)
        pltpu.async_copy(w1_hbm.at[b, pl.ds(s, CH)], w1b, sem)
        pltpu.async_copy(w2_hbm.at[b, pl.ds(s, CH)], w2b, sem)
        pltpu.async_copy(w3_hbm.at[b, pl.ds(s, CH)], w3b, sem)

    def wait_chunk(k, slot, b, c0, single):
        val0b, val1b, idxb, w0b, w1b, w2b, w3b, sem = slots[slot]
        s = k * CH

        @pl.when(jnp.logical_not(single))
        def _():
            pltpu.make_async_copy(
                inp_hbm.at[b * C + c0, pl.ds(s, CH)], val0b, sem).wait()
            pltpu.make_async_copy(
                inp_hbm.at[b * C + c0 + 1, pl.ds(s, CH)], val1b, sem).wait()
        pltpu.make_async_copy(idx_hbm.at[b, pl.ds(s, CH)], idxb, sem).wait()
        pltpu.make_async_copy(w0_hbm.at[b, pl.ds(s, CH)], w0b, sem).wait()
        pltpu.make_async_copy(w1_hbm.at[b, pl.ds(s, CH)], w1b, sem).wait()
        pltpu.make_async_copy(w2_hbm.at[b, pl.ds(s, CH)], w2b, sem).wait()
        pltpu.make_async_copy(w3_hbm.at[b, pl.ds(s, CH)], w3b, sem).wait()

    def compute_chunk(slot):
        val0b, val1b, idxb, w0b, w1b, w2b, w3b, sem = slots[slot]

        @plsc.parallel_loop(0, CH // 16, 1, unroll=8)
        def _g(g):
            o = g * 16
            v0 = val0b[pl.ds(o, 16)]
            v1 = val1b[pl.ds(o, 16)]
            ii = idxb[pl.ds(o, 16)]
            i1 = ii + 1
            i2 = ii + W
            i3 = ii + (W + 1)
            w0 = w0b[pl.ds(o, 16)]
            w1 = w1b[pl.ds(o, 16)]
            w2 = w2b[pl.ds(o, 16)]
            w3 = w3b[pl.ds(o, 16)]
            plsc.addupdate_scatter(acc0, [ii], v0 * w0)
            plsc.addupdate_scatter(acc1, [ii], v1 * w0)
            plsc.addupdate_scatter(acc0, [i1], v0 * w1)
            plsc.addupdate_scatter(acc1, [i1], v1 * w1)
            plsc.addupdate_scatter(acc0, [i2], v0 * w2)
            plsc.addupdate_scatter(acc1, [i2], v1 * w2)
            plsc.addupdate_scatter(acc0, [i3], v0 * w3)
            plsc.addupdate_scatter(acc1, [i3], v1 * w3)

    def unit_loop(t, carry):
        u = wid + t * NW

        @pl.when(u < NUNIT)
        def _():
            bl = ((u >= NUNIT_B).astype(jnp.int32)
                  + (u >= 2 * NUNIT_B).astype(jnp.int32)
                  + (u >= 3 * NUNIT_B).astype(jnp.int32))
            b = bl + boff
            j = u - bl * NUNIT_B
            c0 = 2 * j
            single = j == (NUNIT_B - 1)

            start_chunk(0, 0, b, c0, single)
            start_chunk(1, 1, b, c0, single)

            @pl.when(single)
            def _():
                for slot in range(2):
                    for vb in (slots[slot][0], slots[slot][1]):

                        @plsc.parallel_loop(0, CH // 16, 1, unroll=8)
                        def _o(i):
                            vb[pl.ds(i * 16, 16)] = jnp.ones((16,), jnp.float32)

            @plsc.parallel_loop(0, ACC_LEN // 16, 1, unroll=8)
            def _z(i):
                acc0[pl.ds(i * 16, 16)] = jnp.zeros((16,), jnp.float32)
                acc1[pl.ds(i * 16, 16)] = jnp.zeros((16,), jnp.float32)

            def chunk_loop(kk, carry2):
                for slot in range(2):
                    k = kk * 2 + slot
                    wait_chunk(k, slot, b, c0, single)
                    compute_chunk(slot)

                    @pl.when(k + 2 < NCHUNK)
                    def _():
                        start_chunk(k + 2, slot, b, c0, single)
                return carry2
            lax.fori_loop(0, NCHUNK // 2, chunk_loop, 0)

            img0 = bl * CP1 + c0
            cp0 = pltpu.async_copy(acc0.at[pl.ds(PAD, HW)], out_hbm.at[img0],
                                   sem0)

            @pl.when(jnp.logical_not(single))
            def _():
                pltpu.async_copy(acc1.at[pl.ds(PAD, HW)], out_hbm.at[img0 + 1],
                                 sem1)
            cp0.wait()

            @pl.when(jnp.logical_not(single))
            def _():
                pltpu.make_async_copy(acc1.at[pl.ds(PAD, HW)],
                                      out_hbm.at[img0 + 1], sem1).wait()
        return carry
    lax.fori_loop(0, NT, unit_loop, 0)


@functools.lru_cache(maxsize=2)
def _sc_scatter(boff):
  return functools.partial(
    pl.kernel,
    out_type=jax.ShapeDtypeStruct((NB * CP1, HW), jnp.float32),
    mesh=plsc.VectorSubcoreMesh(core_axis_name="c", subcore_axis_name="s",
                                num_cores=NC, num_subcores=NS),
    compiler_params=pltpu.CompilerParams(needs_layout_passes=False,
                                         use_tc_tiling_on_sc=False),
    scratch_types=(
        [pltpu.VMEM((ACC_LEN,), jnp.float32), pltpu.VMEM((ACC_LEN,), jnp.float32)]
        + [pltpu.VMEM((CH,), jnp.float32),
           pltpu.VMEM((CH,), jnp.float32),
           pltpu.VMEM((CH,), jnp.int32),
           pltpu.VMEM((CH,), jnp.float32),
           pltpu.VMEM((CH,), jnp.float32),
           pltpu.VMEM((CH,), jnp.float32),
           pltpu.VMEM((CH,), jnp.float32)] * 2
        + [pltpu.SemaphoreType.DMA, pltpu.SemaphoreType.DMA]
    ),
  )(functools.partial(_sc_body, boff))


def _norm_body(num_ref, den_ref, out_ref):
    den = den_ref[...]
    den = jnp.where(den == 0.0, 1.0, den)
    out_ref[...] = num_ref[...] / den


CG = 16


def _normalize(acc4):
    return pl.pallas_call(
        _norm_body,
        grid=(NB, C // CG),
        in_specs=[pl.BlockSpec((1, CG, H, W), lambda b, c: (b, c, 0, 0)),
                  pl.BlockSpec((1, 1, H, W), lambda b, c: (b, C, 0, 0))],
        out_specs=pl.BlockSpec((1, CG, H, W), lambda b, c: (b, c, 0, 0)),
        out_shape=jax.ShapeDtypeStruct((NB, C, H, W), jnp.float32),
    )(acc4, acc4)


def kernel(tenInput, tenFlow, tenMetric):
    idx4, w04, w14, w24, w34 = _precompute(tenFlow, tenMetric)
    idx = idx4.reshape(B, HW)
    ws = [w.reshape(B, HW) for w in (w04, w14, w24, w34)]
    inp2d = tenInput.reshape(B * C, HW)
    acc = _sc_scatter(0)(inp2d, idx, *ws)
    return _normalize(acc.reshape(NB, CP1, H, W))


# speedup vs baseline: 1.1238x; 1.0031x over previous
import functools

import jax
import jax.numpy as jnp
from jax import lax
from jax.experimental import pallas as pl
from jax.experimental.pallas import tpu as pltpu
from jax.experimental.pallas import tpu_sc as plsc

B, C, H, W = 4, 96, 224, 224
HW = H * W
CP1 = C + 1
NIMG = B * CP1
PAD = 240
ACC_LEN = HW + 2 * PAD
NC, NS = 2, 16
NW = NC * NS
CH = 1792
NCHUNK = HW // CH
NUNIT_B = 49
NB = 4
NUNIT = NB * NUNIT_B
NT = (NUNIT + NW - 1) // NW


def _pre_body(f0_ref, f1_ref, m_ref, idx_ref, w0_ref, w1_ref, w2_ref, w3_ref):
    shp = (1, 1, H, W)
    xs = lax.broadcasted_iota(jnp.int32, shp, 3).astype(jnp.float32)
    ys = lax.broadcasted_iota(jnp.int32, shp, 2).astype(jnp.float32)
    fx = xs + f0_ref[...]
    fy = ys + f1_ref[...]
    nwx = jnp.floor(fx)
    nwy = jnp.floor(fy)
    ix = nwx.astype(jnp.int32)
    iy = nwy.astype(jnp.int32)
    tx = fx - nwx
    ty = fy - nwy
    ex = jnp.exp(m_ref[...])
    vx0 = (ix >= 0) & (ix < W)
    vx1 = (ix >= -1) & (ix < W - 1)
    vy0 = (iy >= 0) & (iy < H)
    vy1 = (iy >= -1) & (iy < H - 1)
    zero = jnp.zeros(shp, jnp.float32)
    w0_ref[...] = jnp.where(vx0 & vy0, (1.0 - tx) * (1.0 - ty) * ex, zero)
    w1_ref[...] = jnp.where(vx1 & vy0, tx * (1.0 - ty) * ex, zero)
    w2_ref[...] = jnp.where(vx0 & vy1, (1.0 - tx) * ty * ex, zero)
    w3_ref[...] = jnp.where(vx1 & vy1, tx * ty * ex, zero)
    oob = (ix < -1) | (ix >= W) | (iy < -1) | (iy >= H)
    idx_ref[...] = jnp.where(oob, 0, iy * W + ix + PAD)


def _precompute(tenFlow, tenMetric):
    img_spec = lambda ch: pl.BlockSpec((1, 1, H, W), lambda b, ch=ch: (b, ch, 0, 0))
    out4 = jax.ShapeDtypeStruct((B, 1, H, W), jnp.float32)
    return pl.pallas_call(
        _pre_body,
        grid=(B,),
        in_specs=[img_spec(0), img_spec(1), img_spec(0)],
        out_specs=[img_spec(0)] * 5,
        out_shape=[jax.ShapeDtypeStruct((B, 1, H, W), jnp.int32),
                   out4, out4, out4, out4],
    )(tenFlow, tenFlow, tenMetric)


def _sc_body(boff, inp_hbm, idx_hbm, w0_hbm, w1_hbm, w2_hbm, w3_hbm, out_hbm,
             acc0, acc1,
             val0b0, val1b0, idxb0, w0b0, w1b0, w2b0, w3b0,
             val0b1, val1b1, idxb1, w0b1, w1b1, w2b1, w3b1,
             sem0, sem1):
    wid = lax.axis_index("s") * NC + lax.axis_index("c")
    slots = ((val0b0, val1b0, idxb0, w0b0, w1b0, w2b0, w3b0, sem0),
             (val0b1, val1b1, idxb1, w0b1, w1b1, w2b1, w3b1, sem1))

    def start_chunk(k, slot, b, c0, single):
        val0b, val1b, idxb, w0b, w1b, w2b, w3b, sem = slots[slot]
        s = k * CH

        @pl.when(jnp.logical_not(single))
        def _():
            pltpu.async_copy(inp_hbm.at[b * C + c0, pl.ds(s, CH)], val0b, sem)
            pltpu.async_copy(inp_hbm.at[b * C + c0 + 1, pl.ds(s, CH)], val1b, sem)
        pltpu.async_copy(idx_hbm.at[b, pl.ds(s, CH)], idxb, sem)
        pltpu.async_copy(w0_hbm.at[b, pl.ds(s, CH)], w0b, sem)
        pltpu.async_copy(w1_hbm.at[b, pl.ds(s, CH)], w1b, sem)
        pltpu.async_copy(w2_hbm.at[b, pl.ds(s, CH)], w2b, sem)
        pltpu.async_copy(w3_hbm.at[b, pl.ds(s, CH)], w3b, sem)

    def wait_chunk(k, slot, b, c0, single):
        val0b, val1b, idxb, w0b, w1b, w2b, w3b, sem = slots[slot]
        s = k * CH

        @pl.when(jnp.logical_not(single))
        def _():
            pltpu.make_async_copy(
                inp_hbm.at[b * C + c0, pl.ds(s, CH)], val0b, sem).wait()
            pltpu.make_async_copy(
                inp_hbm.at[b * C + c0 + 1, pl.ds(s, CH)], val1b, sem).wait()
        pltpu.make_async_copy(idx_hbm.at[b, pl.ds(s, CH)], idxb, sem).wait()
        pltpu.make_async_copy(w0_hbm.at[b, pl.ds(s, CH)], w0b, sem).wait()
        pltpu.make_async_copy(w1_hbm.at[b, pl.ds(s, CH)], w1b, sem).wait()
        pltpu.make_async_copy(w2_hbm.at[b, pl.ds(s, CH)], w2b, sem).wait()
        pltpu.make_async_copy(w3_hbm.at[b, pl.ds(s, CH)], w3b, sem).wait()

    def compute_chunk(slot):
        val0b, val1b, idxb, w0b, w1b, w2b, w3b, sem = slots[slot]

        @plsc.parallel_loop(0, CH // 16, 1, unroll=8)
        def _g(g):
            o = g * 16
            v0 = val0b[pl.ds(o, 16)]
            v1 = val1b[pl.ds(o, 16)]
            ii = idxb[pl.ds(o, 16)]
            i1 = ii + 1
            i2 = ii + W
            i3 = ii + (W + 1)
            w0 = w0b[pl.ds(o, 16)]
            w1 = w1b[pl.ds(o, 16)]
            w2 = w2b[pl.ds(o, 16)]
            w3 = w3b[pl.ds(o, 16)]
            plsc.addupdate_scatter(acc0, [ii], v0 * w0)
            plsc.addupdate_scatter(acc1, [ii], v1 * w0)
            plsc.addupdate_scatter(acc0, [i1], v0 * w1)
            plsc.addupdate_scatter(acc1, [i1], v1 * w1)
            plsc.addupdate_scatter(acc0, [i2], v0 * w2)
            plsc.addupdate_scatter(acc1, [i2], v1 * w2)
            plsc.addupdate_scatter(acc0, [i3], v0 * w3)
            plsc.addupdate_scatter(acc1, [i3], v1 * w3)

    def unit_loop(t, carry):
        u = wid + t * NW

        @pl.when(u < NUNIT)
        def _():
            bl = ((u >= NUNIT_B).astype(jnp.int32)
                  + (u >= 2 * NUNIT_B).astype(jnp.int32)
                  + (u >= 3 * NUNIT_B).astype(jnp.int32))
            b = bl + boff
            j = u - bl * NUNIT_B
            c0 = 2 * j
            single = j == (NUNIT_B - 1)

            start_chunk(0, 0, b, c0, single)
            start_chunk(1, 1, b, c0, single)

            @pl.when(single)
            def _():
                for slot in range(2):
                    for vb in (slots[slot][0], slots[slot][1]):

                        @plsc.parallel_loop(0, CH // 16, 1, unroll=8)
                        def _o(i):
                            vb[pl.ds(i * 16, 16)] = jnp.ones((16,), jnp.float32)

            @plsc.parallel_loop(0, ACC_LEN // 16, 1, unroll=8)
            def _z(i):
                acc0[pl.ds(i * 16, 16)] = jnp.zeros((16,), jnp.float32)
                acc1[pl.ds(i * 16, 16)] = jnp.zeros((16,), jnp.float32)

            def chunk_loop(kk, carry2):
                for slot in range(2):
                    k = kk * 2 + slot
                    wait_chunk(k, slot, b, c0, single)
                    compute_chunk(slot)

                    @pl.when(k + 2 < NCHUNK)
                    def _():
                        start_chunk(k + 2, slot, b, c0, single)
                return carry2
            lax.fori_loop(0, NCHUNK // 2, chunk_loop, 0)

            img0 = bl * CP1 + c0
            cp0 = pltpu.async_copy(acc0.at[pl.ds(PAD, HW)], out_hbm.at[img0],
                                   sem0)

            @pl.when(jnp.logical_not(single))
            def _():
                pltpu.async_copy(acc1.at[pl.ds(PAD, HW)], out_hbm.at[img0 + 1],
                                 sem1)
            cp0.wait()

            @pl.when(jnp.logical_not(single))
            def _():
                pltpu.make_async_copy(acc1.at[pl.ds(PAD, HW)],
                                      out_hbm.at[img0 + 1], sem1).wait()
        return carry
    lax.fori_loop(0, NT, unit_loop, 0)


@functools.lru_cache(maxsize=2)
def _sc_scatter(boff):
  return functools.partial(
    pl.kernel,
    out_type=jax.ShapeDtypeStruct((NB * CP1, HW), jnp.float32),
    mesh=plsc.VectorSubcoreMesh(core_axis_name="c", subcore_axis_name="s",
                                num_cores=NC, num_subcores=NS),
    compiler_params=pltpu.CompilerParams(needs_layout_passes=False,
                                         use_tc_tiling_on_sc=False),
    scratch_types=(
        [pltpu.VMEM((ACC_LEN,), jnp.float32), pltpu.VMEM((ACC_LEN,), jnp.float32)]
        + [pltpu.VMEM((CH,), jnp.float32),
           pltpu.VMEM((CH,), jnp.float32),
           pltpu.VMEM((CH,), jnp.int32),
           pltpu.VMEM((CH,), jnp.float32),
           pltpu.VMEM((CH,), jnp.float32),
           pltpu.VMEM((CH,), jnp.float32),
           pltpu.VMEM((CH,), jnp.float32)] * 2
        + [pltpu.SemaphoreType.DMA, pltpu.SemaphoreType.DMA]
    ),
  )(functools.partial(_sc_body, boff))


def _norm_body(num_ref, den_ref, out_ref):
    den = den_ref[...]
    den = jnp.where(den == 0.0, 1.0, den)
    out_ref[...] = num_ref[...] / den


CG = 32


def _normalize(acc4):
    return pl.pallas_call(
        _norm_body,
        grid=(NB, C // CG),
        in_specs=[pl.BlockSpec((1, CG, H, W), lambda b, c: (b, c, 0, 0)),
                  pl.BlockSpec((1, 1, H, W), lambda b, c: (b, C, 0, 0))],
        out_specs=pl.BlockSpec((1, CG, H, W), lambda b, c: (b, c, 0, 0)),
        out_shape=jax.ShapeDtypeStruct((NB, C, H, W), jnp.float32),
    )(acc4, acc4)


def kernel(tenInput, tenFlow, tenMetric):
    idx4, w04, w14, w24, w34 = _precompute(tenFlow, tenMetric)
    idx = idx4.reshape(B, HW)
    ws = [w.reshape(B, HW) for w in (w04, w14, w24, w34)]
    inp2d = tenInput.reshape(B * C, HW)
    acc = _sc_scatter(0)(inp2d, idx, *ws)
    return _normalize(acc.reshape(NB, CP1, H, W))
